# Initial kernel scaffold; baseline (speedup 1.0000x reference)
import jax
import jax.numpy as jnp
from jax.experimental import pallas as pl


def kernel(x, W1, b1, W2, b2, W3, b3, Wg, bg):
    raise NotImplementedError("write your pallas kernel here")


# speedup vs baseline: 1.6865x; 1.6865x over previous
import functools

import jax
import jax.numpy as jnp
from jax import lax
from jax.experimental import pallas as pl
from jax.experimental.pallas import tpu as pltpu
from jax.experimental.pallas import tpu_sc as plsc

L, D, H, O, E, K = 2048, 1024, 2048, 1024, 8, 2
T = 256
NS_MAX = L * K // T + E - 1
NTOT = (L * K // T + E) * T
TG = 256
NW = 32
TPW = L // NW


def _gate_kernel(x_ref, wg_ref, bg_ref, meta_ref, aux_ref, acc_ref):
    step = pl.program_id(0)

    @pl.when(step == 0)
    def _():
        acc_ref[...] = jnp.zeros_like(acc_ref)

    logits = lax.dot_general(
        x_ref[...], wg_ref[...], (((1,), (1,)), ((), ())),
        preferred_element_type=jnp.float32) + bg_ref[...]
    lane = lax.broadcasted_iota(jnp.int32, (TG, 128), 1)

    m1 = jnp.max(logits, axis=1, keepdims=True)
    idx1 = jnp.min(jnp.where(logits >= m1, lane, 127), axis=1, keepdims=True)
    a1 = (lane == idx1).astype(jnp.float32)
    masked = jnp.where(lane == idx1, -jnp.inf, logits)
    m2 = jnp.max(masked, axis=1, keepdims=True)
    idx2 = jnp.min(jnp.where(masked >= m2, lane, 127), axis=1, keepdims=True)
    a2 = (lane == idx2).astype(jnp.float32)

    e2 = jnp.exp(m2 - m1)
    w0 = 1.0 / (1.0 + e2)
    w1 = 1.0 - w0

    sel = a1 + a2
    r_io = lax.broadcasted_iota(jnp.int32, (TG, TG), 0)
    c_io = lax.broadcasted_iota(jnp.int32, (TG, TG), 1)
    lt = (c_io < r_io).astype(jnp.float32)
    prefix = lax.dot_general(
        lt, sel, (((1,), (0,)), ((), ())),
        preferred_element_type=jnp.float32) + acc_ref[0:1, :]
    rank0 = jnp.sum(prefix * a1, axis=1, keepdims=True)
    rank1 = jnp.sum(prefix * a2, axis=1, keepdims=True)

    acc_ref[0:1, :] = acc_ref[0:1, :] + jnp.sum(sel, axis=0, keepdims=True)
    wsum = jnp.where(lane == 0, w0, jnp.where(lane == 1, w1, 0.0))
    acc_ref[1:2, :] = acc_ref[1:2, :] + jnp.sum(wsum, axis=0, keepdims=True)

    meta_ref[...] = jnp.where(
        lane == 0, idx1.astype(jnp.float32),
        jnp.where(lane == 1, idx2.astype(jnp.float32),
                  jnp.where(lane == 2, rank0,
                            jnp.where(lane == 3, rank1,
                                      jnp.where(lane == 4, w0,
                                                jnp.where(lane == 5, w1, 0.0))))))

    srow = acc_ref[1:2, :]
    l1 = lax.broadcasted_iota(jnp.int32, (1, 128), 1)
    diff = jnp.sum(jnp.where(l1 == 0, srow, jnp.where(l1 == 1, -srow, 0.0)),
                   axis=1, keepdims=True)
    lb = diff * diff / (2.0 * float(L) * float(L))
    r8 = lax.broadcasted_iota(jnp.int32, (8, 128), 0)
    l8 = lax.broadcasted_iota(jnp.int32, (8, 128), 1)
    aux_ref[...] = jnp.where(r8 == 0, acc_ref[0:1, :],
                             jnp.where((r8 == 1) & (l8 == 0), lb, 0.0))


_gate_call = pl.pallas_call(
    _gate_kernel,
    grid=(L // TG,),
    in_specs=[pl.BlockSpec((TG, D), lambda i: (i, 0)),
              pl.BlockSpec((128, D), lambda i: (0, 0)),
              pl.BlockSpec((1, 128), lambda i: (0, 0))],
    out_specs=[pl.BlockSpec((TG, 128), lambda i: (i, 0)),
               pl.BlockSpec((8, 128), lambda i: (0, 0))],
    out_shape=[jax.ShapeDtypeStruct((L, 128), jnp.float32),
               jax.ShapeDtypeStruct((8, 128), jnp.float32)],
    scratch_shapes=[pltpu.VMEM((8, 128), jnp.float32)],
)


def _stage_kernel(sched_ref, a_ref, w_ref, b_ref, o_ref, *, relu):
    del sched_ref
    h = lax.dot_general(
        a_ref[...], w_ref[0], (((1,), (1,)), ((), ())),
        preferred_element_type=jnp.float32) + b_ref[0]
    o_ref[...] = jnp.maximum(h, 0.0) if relu else h


def _make_stage(d_in, d_out, relu):
    return pl.pallas_call(
        functools.partial(_stage_kernel, relu=relu),
        grid_spec=pltpu.PrefetchScalarGridSpec(
            num_scalar_prefetch=1,
            grid=(NS_MAX,),
            in_specs=[pl.BlockSpec((T, d_in), lambda i, s: (s[1, i], 0)),
                      pl.BlockSpec((1, d_out, d_in), lambda i, s: (s[0, i], 0, 0)),
                      pl.BlockSpec((1, 1, d_out), lambda i, s: (s[0, i], 0, 0))],
            out_specs=pl.BlockSpec((T, d_out), lambda i, s: (s[1, i], 0)),
        ),
        out_shape=jax.ShapeDtypeStruct((NTOT, d_out), jnp.float32),
    )


_stage1 = _make_stage(D, H, True)
_stage2 = _make_stage(H, H, True)
_stage3 = _make_stage(H, O, False)


def _combine_kernel(g0_ref, g1_ref, meta_ref, o_ref):
    w0 = meta_ref[:, 4:5]
    w1 = meta_ref[:, 5:6]
    o_ref[...] = g0_ref[...] * w0 + g1_ref[...] * w1


_combine_call = pl.pallas_call(
    _combine_kernel,
    grid=(L // 256,),
    in_specs=[pl.BlockSpec((256, O), lambda i: (i, 0)),
              pl.BlockSpec((256, O), lambda i: (i, 0)),
              pl.BlockSpec((256, 128), lambda i: (i, 0))],
    out_specs=pl.BlockSpec((256, O), lambda i: (i, 0)),
    out_shape=jax.ShapeDtypeStruct((L, O), jnp.float32),
)


def _pos_kernel(meta_ref, ao_ref, pos_ref):
    lane = lax.broadcasted_iota(jnp.int32, (L, 128), 1)
    idx0 = meta_ref[:, 0:1].astype(jnp.int32)
    idx1 = meta_ref[:, 1:2].astype(jnp.int32)
    r0 = meta_ref[:, 2:3].astype(jnp.int32)
    r1 = meta_ref[:, 3:4].astype(jnp.int32)
    ao_row = ao_ref[...]
    a0 = jnp.sum(jnp.where(lane == idx0, ao_row, 0), axis=1, keepdims=True)
    a1 = jnp.sum(jnp.where(lane == idx1, ao_row, 0), axis=1, keepdims=True)
    pos_ref[...] = jnp.where(lane == 0, a0 + r0,
                             jnp.where(lane == 1, a1 + r1, 0))


_pos_call = pl.pallas_call(
    _pos_kernel,
    in_specs=[pl.BlockSpec((L, 128), lambda: (0, 0)),
              pl.BlockSpec((1, 128), lambda: (0, 0))],
    out_specs=pl.BlockSpec((L, 128), lambda: (0, 0)),
    out_shape=jax.ShapeDtypeStruct((L, 128), jnp.int32),
)


def _dispatch_body(x_hbm, p0_hbm, p1_hbm, xs_hbm, p0_v, p1_v, rows_v, sem):
    wid = lax.axis_index("s") * 2 + lax.axis_index("c")
    base = wid * TPW
    pltpu.sync_copy(p0_hbm.at[pl.ds(base, TPW)], p0_v)
    pltpu.sync_copy(p1_hbm.at[pl.ds(base, TPW)], p1_v)
    pltpu.sync_copy(x_hbm.at[pl.ds(base, TPW)], rows_v)
    pltpu.async_copy(rows_v, xs_hbm.at[p0_v], sem).wait()
    pltpu.async_copy(rows_v, xs_hbm.at[p1_v], sem).wait()


def _gather2_body(ys_hbm, p0_hbm, p1_hbm, g0_hbm, g1_hbm, p_v, rows_v, sem):
    wid = lax.axis_index("s") * 2 + lax.axis_index("c")
    base = wid * TPW
    pltpu.sync_copy(p0_hbm.at[pl.ds(base, TPW)], p_v)
    pltpu.async_copy(ys_hbm.at[p_v], rows_v, sem).wait()
    pltpu.sync_copy(rows_v, g0_hbm.at[pl.ds(base, TPW)])
    pltpu.sync_copy(p1_hbm.at[pl.ds(base, TPW)], p_v)
    pltpu.async_copy(ys_hbm.at[p_v], rows_v, sem).wait()
    pltpu.sync_copy(rows_v, g1_hbm.at[pl.ds(base, TPW)])


@functools.cache
def _sc_kernels():
    mesh = plsc.VectorSubcoreMesh(core_axis_name="c", subcore_axis_name="s")
    dispatch = pl.kernel(
        _dispatch_body,
        out_type=jax.ShapeDtypeStruct((NTOT, D), jnp.float32),
        mesh=mesh,
        scratch_types=[pltpu.VMEM((TPW,), jnp.int32),
                       pltpu.VMEM((TPW,), jnp.int32),
                       pltpu.VMEM((TPW, D), jnp.float32),
                       pltpu.SemaphoreType.DMA],
    )
    gather2 = pl.kernel(
        _gather2_body,
        out_type=[jax.ShapeDtypeStruct((L, O), jnp.float32),
                  jax.ShapeDtypeStruct((L, O), jnp.float32)],
        mesh=mesh,
        scratch_types=[pltpu.VMEM((TPW,), jnp.int32),
                       pltpu.VMEM((TPW, O), jnp.float32),
                       pltpu.SemaphoreType.DMA],
    )
    return dispatch, gather2


def kernel(x, W1, b1, W2, b2, W3, b3, Wg, bg):
    x2 = x.reshape(L, D)
    wg_pad = jnp.zeros((128, D), jnp.float32).at[:E].set(Wg)
    bg_pad = jnp.full((1, 128), -1e30, jnp.float32).at[0, :E].set(bg)

    meta, aux = _gate_call(x2, wg_pad, bg_pad)
    counts = aux[0, :E].astype(jnp.int32)
    lb = aux[1, 0]

    tiles_e = (counts + T - 1) // T
    cum_t = jnp.cumsum(tiles_e)
    cum_excl = cum_t - tiles_e
    ao128 = jnp.zeros((1, 128), jnp.int32).at[0, :E].set(
        (cum_excl * T).astype(jnp.int32))
    s_ids = jnp.minimum(jnp.arange(NS_MAX, dtype=jnp.int32), cum_t[-1] - 1)
    e_s = jnp.searchsorted(cum_t, s_ids, side="right").astype(jnp.int32)
    sched = jnp.stack([e_s, s_ids])

    pos = _pos_call(meta, ao128)
    pos0 = pos[:, 0]
    pos1 = pos[:, 1]
    dispatch, gather2 = _sc_kernels()
    xs = dispatch(x2, pos0, pos1)
    h1 = _stage1(sched, xs, W1, b1.reshape(E, 1, H))
    h2 = _stage2(sched, h1, W2, b2.reshape(E, 1, H))
    ys = _stage3(sched, h2, W3, b3.reshape(E, 1, O))
    g0, g1 = gather2(ys, pos0, pos1)
    out = _combine_call(g0, g1, meta)
    return out.reshape(1, L, O), lb.reshape(())


# speedup vs baseline: 1.7150x; 1.0169x over previous
import functools

import jax
import jax.numpy as jnp
from jax import lax
from jax.experimental import pallas as pl
from jax.experimental.pallas import tpu as pltpu
from jax.experimental.pallas import tpu_sc as plsc

L, D, H, O, E, K = 2048, 1024, 2048, 1024, 8, 2
T = 256
NS_MAX = L * K // T + E - 1
NTOT = (L * K // T + E) * T
TG = 256
NW = 32
TPW = L // NW


def _gate_kernel(x_ref, wg_ref, bg_ref, meta_ref, aux_ref, acc_ref):
    step = pl.program_id(0)

    @pl.when(step == 0)
    def _():
        acc_ref[...] = jnp.zeros_like(acc_ref)

    logits = lax.dot_general(
        x_ref[...], wg_ref[...], (((1,), (1,)), ((), ())),
        preferred_element_type=jnp.float32) + bg_ref[...]
    lane8 = lax.broadcasted_iota(jnp.int32, (TG, E), 1)
    lane = lax.broadcasted_iota(jnp.int32, (TG, 128), 1)

    m1 = jnp.max(logits, axis=1, keepdims=True)
    idx1 = jnp.min(jnp.where(logits >= m1, lane8, E), axis=1, keepdims=True)
    a1 = (lane8 == idx1).astype(jnp.float32)
    masked = jnp.where(lane8 == idx1, -jnp.inf, logits)
    m2 = jnp.max(masked, axis=1, keepdims=True)
    idx2 = jnp.min(jnp.where(masked >= m2, lane8, E), axis=1, keepdims=True)
    a2 = (lane8 == idx2).astype(jnp.float32)

    e2 = jnp.exp(m2 - m1)
    w0 = 1.0 / (1.0 + e2)
    w1 = 1.0 - w0

    sel = a1 + a2
    r_io = lax.broadcasted_iota(jnp.int32, (TG, TG), 0)
    c_io = lax.broadcasted_iota(jnp.int32, (TG, TG), 1)
    lt = (c_io < r_io).astype(jnp.float32)
    prefix = lax.dot_general(
        lt, sel, (((1,), (0,)), ((), ())),
        preferred_element_type=jnp.float32) + acc_ref[0:1, 0:E]
    rank0 = jnp.sum(prefix * a1, axis=1, keepdims=True)
    rank1 = jnp.sum(prefix * a2, axis=1, keepdims=True)

    upd = jnp.concatenate(
        [jnp.sum(sel, axis=0, keepdims=True), jnp.zeros((1, 128 - E))], axis=1)
    acc_ref[0:1, :] = acc_ref[0:1, :] + upd
    wsum = jnp.where(lane == 0, w0, jnp.where(lane == 1, w1, 0.0))
    acc_ref[1:2, :] = acc_ref[1:2, :] + jnp.sum(wsum, axis=0, keepdims=True)

    meta_ref[...] = jnp.where(
        lane == 0, idx1.astype(jnp.float32),
        jnp.where(lane == 1, idx2.astype(jnp.float32),
                  jnp.where(lane == 2, rank0,
                            jnp.where(lane == 3, rank1,
                                      jnp.where(lane == 4, w0,
                                                jnp.where(lane == 5, w1, 0.0))))))

    srow = acc_ref[1:2, :]
    l1 = lax.broadcasted_iota(jnp.int32, (1, 128), 1)
    diff = jnp.sum(jnp.where(l1 == 0, srow, jnp.where(l1 == 1, -srow, 0.0)),
                   axis=1, keepdims=True)
    lb = diff * diff / (2.0 * float(L) * float(L))
    r8 = lax.broadcasted_iota(jnp.int32, (8, 128), 0)
    l8 = lax.broadcasted_iota(jnp.int32, (8, 128), 1)
    aux_ref[...] = jnp.where(r8 == 0, acc_ref[0:1, :],
                             jnp.where((r8 == 1) & (l8 == 0), lb, 0.0))


_gate_call = pl.pallas_call(
    _gate_kernel,
    grid=(L // TG,),
    in_specs=[pl.BlockSpec((TG, D), lambda i: (i, 0)),
              pl.BlockSpec((E, D), lambda i: (0, 0)),
              pl.BlockSpec((1, E), lambda i: (0, 0))],
    out_specs=[pl.BlockSpec((TG, 128), lambda i: (i, 0)),
               pl.BlockSpec((8, 128), lambda i: (0, 0))],
    out_shape=[jax.ShapeDtypeStruct((L, 128), jnp.float32),
               jax.ShapeDtypeStruct((8, 128), jnp.float32)],
    scratch_shapes=[pltpu.VMEM((8, 128), jnp.float32)],
)


def _stage_kernel(sched_ref, a_ref, w_ref, b_ref, o_ref, *, relu):
    del sched_ref
    h = lax.dot_general(
        a_ref[...], w_ref[0], (((1,), (1,)), ((), ())),
        preferred_element_type=jnp.float32) + b_ref[0]
    o_ref[...] = jnp.maximum(h, 0.0) if relu else h


def _make_stage(d_in, d_out, relu):
    return pl.pallas_call(
        functools.partial(_stage_kernel, relu=relu),
        grid_spec=pltpu.PrefetchScalarGridSpec(
            num_scalar_prefetch=1,
            grid=(NS_MAX,),
            in_specs=[pl.BlockSpec((T, d_in), lambda i, s: (s[1, i], 0)),
                      pl.BlockSpec((1, d_out, d_in), lambda i, s: (s[0, i], 0, 0)),
                      pl.BlockSpec((1, 1, d_out), lambda i, s: (s[0, i], 0, 0))],
            out_specs=pl.BlockSpec((T, d_out), lambda i, s: (s[1, i], 0)),
        ),
        out_shape=jax.ShapeDtypeStruct((NTOT, d_out), jnp.float32),
    )


_stage1 = _make_stage(D, H, True)
_stage2 = _make_stage(H, H, True)
_stage3 = _make_stage(H, O, False)


def _combine_kernel(g0_ref, g1_ref, meta_ref, o_ref):
    w0 = meta_ref[:, 4:5]
    w1 = meta_ref[:, 5:6]
    o_ref[...] = g0_ref[...] * w0 + g1_ref[...] * w1


_combine_call = pl.pallas_call(
    _combine_kernel,
    grid=(L // 256,),
    in_specs=[pl.BlockSpec((256, O), lambda i: (i, 0)),
              pl.BlockSpec((256, O), lambda i: (i, 0)),
              pl.BlockSpec((256, 128), lambda i: (i, 0))],
    out_specs=pl.BlockSpec((256, O), lambda i: (i, 0)),
    out_shape=jax.ShapeDtypeStruct((L, O), jnp.float32),
)


def _pos_kernel(meta_ref, aux_ref, pos_ref, sched_ref):
    lane1 = lax.broadcasted_iota(jnp.int32, (1, 128), 1).astype(jnp.float32)
    counts = aux_ref[0:1, :]
    tiles = jnp.floor((counts + float(T - 1)) * (1.0 / T))
    r_io = lax.broadcasted_iota(jnp.int32, (128, 128), 0)
    c_io = lax.broadcasted_iota(jnp.int32, (128, 128), 1)
    le = (r_io <= c_io).astype(jnp.float32)
    cum = lax.dot_general(tiles, le, (((1,), (0,)), ((), ())),
                          preferred_element_type=jnp.float32)
    ao_row = (cum - tiles) * float(T)
    ns = jnp.sum(jnp.where(lane1 == float(E - 1), cum, 0.0),
                 axis=1, keepdims=True)
    s_clamp = jnp.minimum(lane1, ns - 1.0)
    eye = (r_io == c_io).astype(jnp.float32)
    cum_col = lax.dot_general(eye, cum, (((1,), (1,)), ((), ())),
                              preferred_element_type=jnp.float32)
    e_row = jnp.sum((cum_col <= s_clamp).astype(jnp.float32),
                    axis=0, keepdims=True)
    r8 = lax.broadcasted_iota(jnp.int32, (8, 128), 0)
    sched_ref[...] = jnp.where(
        r8 == 0, e_row, jnp.where(r8 == 1, s_clamp, 0.0)).astype(jnp.int32)

    lane = lax.broadcasted_iota(jnp.int32, (L, 128), 1)
    idx0 = meta_ref[:, 0:1].astype(jnp.int32)
    idx1 = meta_ref[:, 1:2].astype(jnp.int32)
    r0 = meta_ref[:, 2:3].astype(jnp.int32)
    r1 = meta_ref[:, 3:4].astype(jnp.int32)
    ao_i = ao_row.astype(jnp.int32)
    a0 = jnp.sum(jnp.where(lane == idx0, ao_i, 0), axis=1, keepdims=True)
    a1 = jnp.sum(jnp.where(lane == idx1, ao_i, 0), axis=1, keepdims=True)
    pos_ref[...] = jnp.where(lane == 0, a0 + r0,
                             jnp.where(lane == 1, a1 + r1, 0))


_pos_call = pl.pallas_call(
    _pos_kernel,
    in_specs=[pl.BlockSpec((L, 128), lambda: (0, 0)),
              pl.BlockSpec((8, 128), lambda: (0, 0))],
    out_specs=[pl.BlockSpec((L, 128), lambda: (0, 0)),
               pl.BlockSpec((8, 128), lambda: (0, 0))],
    out_shape=[jax.ShapeDtypeStruct((L, 128), jnp.int32),
               jax.ShapeDtypeStruct((8, 128), jnp.int32)],
)


def _dispatch_body(x_hbm, p0_hbm, p1_hbm, xs_hbm, p0_v, p1_v, rows_v,
                   s1, s2, s3):
    wid = lax.axis_index("s") * 2 + lax.axis_index("c")
    base = wid * TPW
    c1 = pltpu.async_copy(p0_hbm.at[pl.ds(base, TPW)], p0_v, s1)
    c2 = pltpu.async_copy(p1_hbm.at[pl.ds(base, TPW)], p1_v, s2)
    c3 = pltpu.async_copy(x_hbm.at[pl.ds(base, TPW)], rows_v, s3)
    c1.wait()
    c3.wait()
    w1 = pltpu.async_copy(rows_v, xs_hbm.at[p0_v], s1)
    c2.wait()
    w2 = pltpu.async_copy(rows_v, xs_hbm.at[p1_v], s2)
    w1.wait()
    w2.wait()


def _gather2_body(ys_hbm, p0_hbm, p1_hbm, g0_hbm, g1_hbm,
                  pa_v, pb_v, a_v, b_v, s1, s2):
    wid = lax.axis_index("s") * 2 + lax.axis_index("c")
    base = wid * TPW
    hpw = TPW // 2
    pvs, bufs, sems = (pa_v, pb_v), (a_v, b_v), (s1, s2)
    srcs = ((p0_hbm, g0_hbm), (p1_hbm, g1_hbm))
    wbs = [None] * 4
    for k in range(4):
        slot, h = divmod(k, 2)
        psrc, gdst = srcs[slot]
        pv, buf, sem = pvs[k % 2], bufs[k % 2], sems[k % 2]
        if k >= 2:
            wbs[k - 2].wait()
        pltpu.async_copy(psrc.at[pl.ds(base + h * hpw, hpw)], pv, sem).wait()
        pltpu.async_copy(ys_hbm.at[pv], buf, sem).wait()
        wbs[k] = pltpu.async_copy(buf, gdst.at[pl.ds(base + h * hpw, hpw)], sem)
    wbs[2].wait()
    wbs[3].wait()


@functools.cache
def _sc_kernels():
    mesh = plsc.VectorSubcoreMesh(core_axis_name="c", subcore_axis_name="s")
    dispatch = pl.kernel(
        _dispatch_body,
        out_type=jax.ShapeDtypeStruct((NTOT, D), jnp.float32),
        mesh=mesh,
        scratch_types=[pltpu.VMEM((TPW,), jnp.int32),
                       pltpu.VMEM((TPW,), jnp.int32),
                       pltpu.VMEM((TPW, D), jnp.float32),
                       pltpu.SemaphoreType.DMA,
                       pltpu.SemaphoreType.DMA,
                       pltpu.SemaphoreType.DMA],
    )
    gather2 = pl.kernel(
        _gather2_body,
        out_type=[jax.ShapeDtypeStruct((L, O), jnp.float32),
                  jax.ShapeDtypeStruct((L, O), jnp.float32)],
        mesh=mesh,
        scratch_types=[pltpu.VMEM((TPW // 2,), jnp.int32),
                       pltpu.VMEM((TPW // 2,), jnp.int32),
                       pltpu.VMEM((TPW // 2, O), jnp.float32),
                       pltpu.VMEM((TPW // 2, O), jnp.float32),
                       pltpu.SemaphoreType.DMA,
                       pltpu.SemaphoreType.DMA],
    )
    return dispatch, gather2


def kernel(x, W1, b1, W2, b2, W3, b3, Wg, bg):
    x2 = x.reshape(L, D)
    meta, aux = _gate_call(x2, Wg, bg.reshape(1, E))
    lb = aux[1, 0]

    pos, sched = _pos_call(meta, aux)
    pos0 = pos[:, 0]
    pos1 = pos[:, 1]
    dispatch, gather2 = _sc_kernels()
    xs = dispatch(x2, pos0, pos1)
    h1 = _stage1(sched, xs, W1, b1.reshape(E, 1, H))
    h2 = _stage2(sched, h1, W2, b2.reshape(E, 1, H))
    ys = _stage3(sched, h2, W3, b3.reshape(E, 1, O))
    g0, g1 = gather2(ys, pos0, pos1)
    out = _combine_call(g0, g1, meta)
    return out.reshape(1, L, O), lb.reshape(())
